# baseline (speedup 1.0000x reference)
import functools

import jax
import jax.numpy as jnp
from jax import lax
from jax.experimental import pallas as pl
from jax.experimental.pallas import tpu as pltpu
from jax.experimental.pallas import tpu_sc as plsc

DMODEL = 64
SCALE = 8.0
C = 128
NBUF = 4
NCOL = 200


def _make_sc_embed(nw, nc):
    mesh = plsc.VectorSubcoreMesh(core_axis_name="c", subcore_axis_name="s")

    @functools.partial(
        pl.kernel,
        mesh=mesh,
        compiler_params=pltpu.CompilerParams(
            use_tc_tiling_on_sc=False, needs_layout_passes=False),
        out_type=jax.ShapeDtypeStruct((NCOL, 8, nw, 8, C), jnp.float32),
        scratch_types=(
            [pltpu.VMEM((NCOL // 8, 1, 8, C), jnp.int32)]
            + [pltpu.VMEM((C, DMODEL), jnp.float32) for _ in range(NBUF)]
            + [pltpu.VMEM((8, 1, 8, C), jnp.float32) for _ in range(NBUF)]
            + [pltpu.SemaphoreType.DMA for _ in range(2 * NBUF)]
        ),
    )
    def emb(x4_hbm, table_hbm, out_hbm, idxv, *rest):
        gbufs = rest[:NBUF]
        tbufs = rest[NBUF:2 * NBUF]
        gsems = rest[2 * NBUF:3 * NBUF]
        osems = rest[3 * NBUF:]
        wid = lax.axis_index("s") * nc + lax.axis_index("c")

        pltpu.sync_copy(x4_hbm.at[:, pl.ds(wid, 1)], idxv)

        ci = lax.iota(jnp.int32, 16)
        zi = ci * 0
        c1s = [(ci + 16 * q) >> 3 for q in range(4)]
        c2s = [(ci + 16 * q) & 7 for q in range(4)]

        def compute(gbuf, tbuf):
            def row(r, acc):
                rv = zi + r
                for q in range(4):
                    v = gbuf[r, pl.ds(16 * q, 16)] * SCALE
                    plsc.store_scatter(tbuf, [c1s[q], zi, c2s[q], rv], v)
                return acc
            lax.fori_loop(0, C, row, 0)

        for b in range(NBUF):
            pltpu.async_copy(
                table_hbm.at[idxv.at[b // 8, 0, b % 8]], gbufs[b], gsems[b])

        def step(jj, carry):
            jb = jj * NBUF
            for b in range(NBUF):
                j = jb + b
                pltpu.make_async_copy(
                    table_hbm.at[idxv.at[j // 8, 0, j % 8]],
                    gbufs[b], gsems[b]).wait()

                @pl.when(j >= NBUF)
                def _(b=b):
                    pltpu.make_async_copy(
                        tbufs[b], out_hbm.at[0, :, pl.ds(wid, 1)],
                        osems[b]).wait()

                compute(gbufs[b], tbufs[b])

                @pl.when(j + NBUF < NCOL)
                def _(b=b, j=j):
                    k = j + NBUF
                    pltpu.async_copy(
                        table_hbm.at[idxv.at[k // 8, 0, k % 8]],
                        gbufs[b], gsems[b])

                pltpu.async_copy(
                    tbufs[b], out_hbm.at[j, :, pl.ds(wid, 1)], osems[b])
            return carry

        lax.fori_loop(0, NCOL // NBUF, step, 0)

        for b in range(NBUF):
            pltpu.make_async_copy(
                tbufs[b], out_hbm.at[0, :, pl.ds(wid, 1)], osems[b]).wait()

    return emb


def kernel(x, table):
    S, T = x.shape
    info = plsc.get_sparse_core_info()
    nc, ns = info.num_cores, info.num_subcores
    nw = nc * ns
    x4 = x.astype(jnp.int32).T.reshape(T // 8, 8, S // C, C).transpose(0, 2, 1, 3)
    out5 = _make_sc_embed(nw, nc)(x4, table)
    return out5.transpose(2, 4, 0, 1, 3).reshape(S, T, DMODEL)


# speedup vs baseline: 1.0028x; 1.0028x over previous
import functools

import jax
import jax.numpy as jnp
from jax import lax
from jax.experimental import pallas as pl
from jax.experimental.pallas import tpu as pltpu
from jax.experimental.pallas import tpu_sc as plsc

DMODEL = 64
SCALE = 8.0
C = 128
NBUF = 4
NCOL = 200


def _make_sc_embed(nw, nc):
    mesh = plsc.VectorSubcoreMesh(core_axis_name="c", subcore_axis_name="s")

    @functools.partial(
        pl.kernel,
        mesh=mesh,
        compiler_params=pltpu.CompilerParams(
            use_tc_tiling_on_sc=False, needs_layout_passes=False),
        out_type=jax.ShapeDtypeStruct((NCOL, 8, nw, 8 * C), jnp.float32),
        scratch_types=(
            [pltpu.VMEM((NCOL // 8, 1, 8, C), jnp.int32)]
            + [pltpu.VMEM((C, DMODEL), jnp.float32) for _ in range(NBUF)]
            + [pltpu.VMEM((8, 8 * C), jnp.float32) for _ in range(NBUF)]
            + [pltpu.SemaphoreType.DMA for _ in range(2 * NBUF)]
        ),
    )
    def emb(x4_hbm, table_hbm, out_hbm, idxv, *rest):
        gbufs = rest[:NBUF]
        tbufs = rest[NBUF:2 * NBUF]
        gsems = rest[2 * NBUF:3 * NBUF]
        osems = rest[3 * NBUF:]
        wid = lax.axis_index("s") * nc + lax.axis_index("c")

        pltpu.sync_copy(x4_hbm.at[:, pl.ds(wid, 1)], idxv)

        ci = lax.iota(jnp.int32, 16)
        zi = ci * 0
        rowc = [(ci + 16 * q) >> 3 for q in range(4)]
        colb = [((ci + 16 * q) & 7) * C for q in range(4)]

        def compute(gbuf, tbuf):
            def rows(i, acc):
                r = i * 2
                for u in range(2):
                    rv = zi + (r + u)
                    for q in range(4):
                        v = gbuf[r + u, pl.ds(16 * q, 16)] * SCALE
                        plsc.store_scatter(tbuf, [rowc[q], colb[q] + rv], v)
                return acc
            lax.fori_loop(0, C // 2, rows, 0)

        def out_write(j, b):
            for c1 in range(8):
                pltpu.async_copy(
                    tbufs[b].at[c1], out_hbm.at[j, c1, wid], osems[b])

        def out_wait(b):
            for c1 in range(8):
                pltpu.make_async_copy(
                    tbufs[b].at[c1], out_hbm.at[0, c1, 0], osems[b]).wait()

        for b in range(NBUF):
            pltpu.async_copy(
                table_hbm.at[idxv.at[b // 8, 0, b % 8]], gbufs[b], gsems[b])

        def step(jj, carry):
            jb = jj * NBUF
            for b in range(NBUF):
                j = jb + b
                pltpu.make_async_copy(
                    table_hbm.at[idxv.at[j // 8, 0, j % 8]],
                    gbufs[b], gsems[b]).wait()

                @pl.when(j >= NBUF)
                def _(b=b):
                    out_wait(b)

                compute(gbufs[b], tbufs[b])

                @pl.when(j + NBUF < NCOL)
                def _(b=b, j=j):
                    k = j + NBUF
                    pltpu.async_copy(
                        table_hbm.at[idxv.at[k // 8, 0, k % 8]],
                        gbufs[b], gsems[b])

                out_write(j, b)
            return carry

        lax.fori_loop(0, NCOL // NBUF, step, 0)

        for b in range(NBUF):
            out_wait(b)

    return emb


def kernel(x, table):
    S, T = x.shape
    info = plsc.get_sparse_core_info()
    nc, ns = info.num_cores, info.num_subcores
    nw = nc * ns
    x4 = x.astype(jnp.int32).T.reshape(T // 8, 8, S // C, C).transpose(0, 2, 1, 3)
    out4 = _make_sc_embed(nw, nc)(x4, table)
    out5 = out4.reshape(NCOL, 8, nw, 8, C)
    return out5.transpose(2, 4, 0, 1, 3).reshape(S, T, DMODEL)


# speedup vs baseline: 1.3205x; 1.3168x over previous
import functools

import jax
import jax.numpy as jnp
from jax import lax
from jax.experimental import pallas as pl
from jax.experimental.pallas import tpu as pltpu
from jax.experimental.pallas import tpu_sc as plsc

DMODEL = 64
SCALE = 8.0
C = 128
NBUF = 4
NCOL = 200


def _make_sc_embed(nw, nc):
    mesh = plsc.VectorSubcoreMesh(core_axis_name="c", subcore_axis_name="s")

    @functools.partial(
        pl.kernel,
        mesh=mesh,
        compiler_params=pltpu.CompilerParams(
            use_tc_tiling_on_sc=False, needs_layout_passes=False),
        out_type=jax.ShapeDtypeStruct((NCOL, 8, nw, 8 * C), jnp.float32),
        scratch_types=(
            [pltpu.VMEM((NCOL // 8, 1, 8, C), jnp.int32)]
            + [pltpu.VMEM((C, DMODEL), jnp.float32) for _ in range(NBUF)]
            + [pltpu.VMEM((8, 8 * C), jnp.float32) for _ in range(NBUF)]
            + [pltpu.SemaphoreType.DMA for _ in range(2 * NBUF)]
        ),
    )
    def emb(x4_hbm, table_hbm, out_hbm, idxv, *rest):
        gbufs = rest[:NBUF]
        tbufs = rest[NBUF:2 * NBUF]
        gsems = rest[2 * NBUF:3 * NBUF]
        osems = rest[3 * NBUF:]
        wid = lax.axis_index("s") * nc + lax.axis_index("c")

        pltpu.sync_copy(x4_hbm.at[:, pl.ds(wid, 1)], idxv)

        ci = lax.iota(jnp.int32, 16)
        zi = ci * 0
        rowc = [(ci + 16 * q) >> 3 for q in range(4)]
        colb = [((ci + 16 * q) & 7) * C for q in range(4)]

        def compute(gbuf, tbuf):
            @plsc.parallel_loop(0, C, 1, unroll=4)
            def _rows(r):
                rv = zi + r
                for q in range(4):
                    v = gbuf[r, pl.ds(16 * q, 16)] * SCALE
                    plsc.store_scatter(tbuf, [rowc[q], colb[q] + rv], v)

        def out_write(j, b):
            for c1 in range(8):
                pltpu.async_copy(
                    tbufs[b].at[c1], out_hbm.at[j, c1, wid], osems[b])

        def out_wait(b):
            for c1 in range(8):
                pltpu.make_async_copy(
                    tbufs[b].at[c1], out_hbm.at[0, c1, 0], osems[b]).wait()

        for b in range(NBUF):
            pltpu.async_copy(
                table_hbm.at[idxv.at[b // 8, 0, b % 8]], gbufs[b], gsems[b])

        def step(jj, carry):
            jb = jj * NBUF
            for b in range(NBUF):
                j = jb + b
                pltpu.make_async_copy(
                    table_hbm.at[idxv.at[j // 8, 0, j % 8]],
                    gbufs[b], gsems[b]).wait()

                @pl.when(j >= NBUF)
                def _(b=b):
                    out_wait(b)

                compute(gbufs[b], tbufs[b])

                @pl.when(j + NBUF < NCOL)
                def _(b=b, j=j):
                    k = j + NBUF
                    pltpu.async_copy(
                        table_hbm.at[idxv.at[k // 8, 0, k % 8]],
                        gbufs[b], gsems[b])

                out_write(j, b)
            return carry

        lax.fori_loop(0, NCOL // NBUF, step, 0)

        for b in range(NBUF):
            out_wait(b)

    return emb


def kernel(x, table):
    S, T = x.shape
    info = plsc.get_sparse_core_info()
    nc, ns = info.num_cores, info.num_subcores
    nw = nc * ns
    x4 = x.astype(jnp.int32).T.reshape(T // 8, 8, S // C, C).transpose(0, 2, 1, 3)
    out4 = _make_sc_embed(nw, nc)(x4, table)
    out5 = out4.reshape(NCOL, 8, nw, 8, C)
    return out5.transpose(2, 4, 0, 1, 3).reshape(S, T, DMODEL)


# speedup vs baseline: 1.3228x; 1.0017x over previous
import functools

import jax
import jax.numpy as jnp
from jax import lax
from jax.experimental import pallas as pl
from jax.experimental.pallas import tpu as pltpu
from jax.experimental.pallas import tpu_sc as plsc

DMODEL = 64
SCALE = 8.0
C = 128
NBUF = 4
NCOL = 200


def _make_sc_embed(nw, nc):
    mesh = plsc.VectorSubcoreMesh(core_axis_name="c", subcore_axis_name="s")

    @functools.partial(
        pl.kernel,
        mesh=mesh,
        compiler_params=pltpu.CompilerParams(
            use_tc_tiling_on_sc=False, needs_layout_passes=False,
            skip_device_barrier=True, disable_semaphore_checks=True,
            disable_bounds_checks=True),
        out_type=jax.ShapeDtypeStruct((NCOL, 8, nw, 8 * C), jnp.float32),
        scratch_types=(
            [pltpu.VMEM((NCOL // 8, 1, 8, C), jnp.int32)]
            + [pltpu.VMEM((C, DMODEL), jnp.float32) for _ in range(NBUF)]
            + [pltpu.VMEM((8, 8 * C), jnp.float32) for _ in range(NBUF)]
            + [pltpu.SemaphoreType.DMA for _ in range(2 * NBUF)]
        ),
    )
    def emb(x4_hbm, table_hbm, out_hbm, idxv, *rest):
        gbufs = rest[:NBUF]
        tbufs = rest[NBUF:2 * NBUF]
        gsems = rest[2 * NBUF:3 * NBUF]
        osems = rest[3 * NBUF:]
        wid = lax.axis_index("s") * nc + lax.axis_index("c")

        pltpu.sync_copy(x4_hbm.at[:, pl.ds(wid, 1)], idxv)

        ci = lax.iota(jnp.int32, 16)
        zi = ci * 0
        rowc = [(ci + 16 * q) >> 3 for q in range(4)]
        colb = [((ci + 16 * q) & 7) * C for q in range(4)]

        def compute(gbuf, tbuf):
            @plsc.parallel_loop(0, C, 1, unroll=4)
            def _rows(r):
                rv = zi + r
                for q in range(4):
                    v = gbuf[r, pl.ds(16 * q, 16)] * SCALE
                    plsc.store_scatter(tbuf, [rowc[q], colb[q] + rv], v)

        def out_write(j, b):
            for c1 in range(8):
                pltpu.async_copy(
                    tbufs[b].at[c1], out_hbm.at[j, c1, wid], osems[b])

        def out_wait(b):
            for c1 in range(8):
                pltpu.make_async_copy(
                    tbufs[b].at[c1], out_hbm.at[0, c1, 0], osems[b]).wait()

        for b in range(NBUF):
            pltpu.async_copy(
                table_hbm.at[idxv.at[b // 8, 0, b % 8]], gbufs[b], gsems[b])

        def step(jj, carry):
            jb = jj * NBUF
            for b in range(NBUF):
                j = jb + b
                pltpu.make_async_copy(
                    table_hbm.at[idxv.at[j // 8, 0, j % 8]],
                    gbufs[b], gsems[b]).wait()

                @pl.when(j >= NBUF)
                def _(b=b):
                    out_wait(b)

                compute(gbufs[b], tbufs[b])

                @pl.when(j + NBUF < NCOL)
                def _(b=b, j=j):
                    k = j + NBUF
                    pltpu.async_copy(
                        table_hbm.at[idxv.at[k // 8, 0, k % 8]],
                        gbufs[b], gsems[b])

                out_write(j, b)
            return carry

        lax.fori_loop(0, NCOL // NBUF, step, 0)

        for b in range(NBUF):
            out_wait(b)

    return emb


def kernel(x, table):
    S, T = x.shape
    info = plsc.get_sparse_core_info()
    nc, ns = info.num_cores, info.num_subcores
    nw = nc * ns
    x4 = x.astype(jnp.int32).T.reshape(T // 8, 8, S // C, C).transpose(0, 2, 1, 3)
    out4 = _make_sc_embed(nw, nc)(x4, table)
    out5 = out4.reshape(NCOL, 8, nw, 8, C)
    return out5.transpose(2, 4, 0, 1, 3).reshape(S, T, DMODEL)
